# initial kernel scaffold (unmeasured)
import jax
import jax.numpy as jnp
from jax import lax
from jax.experimental import pallas as pl
from jax.experimental.pallas import tpu as pltpu

N_DEV = 4
HQ_LOC = 4
DH = 64
SQ_LOC = 256
B = 2
D_MODEL = 512
D_HID = 1024


def kernel(x, Wq, K_ext, V_ext, Wo):
    x2 = x.reshape(B * SQ_LOC, D_MODEL).astype(jnp.bfloat16)
    payload = jnp.concatenate(
        [Wq.astype(jnp.bfloat16), Wo.T.astype(jnp.bfloat16)], axis=0
    )
    kt = jnp.transpose(K_ext, (2, 0, 1, 3)).reshape(16, B * SQ_LOC, DH)
    vt = jnp.transpose(V_ext, (2, 0, 1, 3)).reshape(16, B * SQ_LOC, DH)
    kt = kt.astype(jnp.bfloat16)
    vt = vt.astype(jnp.bfloat16)

    n_rows = B * SQ_LOC
    hid_loc = HQ_LOC * DH

    def body(x_ref, pay_ref, k_ref, v_ref, out_ref,
             comm_ref, send_sems, recv_sems):
        my = lax.axis_index("i")
        left = lax.rem(my + (N_DEV - 1), N_DEV)
        right = lax.rem(my + 1, N_DEV)

        barrier = pltpu.get_barrier_semaphore()
        for nbr in (left, right):
            pl.semaphore_signal(
                barrier, inc=1,
                device_id=(nbr,), device_id_type=pl.DeviceIdType.MESH,
            )
        pl.semaphore_wait(barrier, 2)

        comm_ref[0] = pay_ref[...]

        row = lax.broadcasted_iota(jnp.int32, (n_rows, n_rows), 0)
        col = lax.broadcasted_iota(jnp.int32, (n_rows, n_rows), 1)
        mask = ((row // SQ_LOC) == (col // SQ_LOC)) & (
            ((row // DH) % HQ_LOC) == ((col // DH) % HQ_LOC)
        )

        for h in range(N_DEV):
            if h < N_DEV - 1:
                rdma = pltpu.make_async_remote_copy(
                    src_ref=comm_ref.at[h],
                    dst_ref=comm_ref.at[h + 1],
                    send_sem=send_sems.at[h],
                    recv_sem=recv_sems.at[h + 1],
                    device_id=(right,),
                    device_id_type=pl.DeviceIdType.MESH,
                )
                rdma.start()

            origin = lax.rem(my - h + N_DEV, N_DEV)
            wq_c = comm_ref[h, :D_MODEL, :]
            woT_c = comm_ref[h, D_MODEL:, :]

            q_full = jnp.dot(
                x_ref[...], wq_c, preferred_element_type=jnp.float32
            )

            ctx_parts = []
            for t in range(HQ_LOC):
                head = origin * HQ_LOC + t
                q = q_full[:, t * DH:(t + 1) * DH].astype(jnp.bfloat16)
                k = k_ref[head]
                s = lax.dot_general(
                    q, k, (((1,), (1,)), ((), ())),
                    preferred_element_type=jnp.float32,
                ) * 0.125
                s = jnp.where(mask, s, -1e9)
                m = jnp.max(s, axis=1, keepdims=True)
                w = jnp.exp(s - m)
                w = w / jnp.sum(w, axis=1, keepdims=True)
                ctx_parts.append(
                    jnp.dot(
                        w.astype(jnp.bfloat16), v_ref[head],
                        preferred_element_type=jnp.float32,
                    )
                )
            ctx = jnp.concatenate(ctx_parts, axis=1).astype(jnp.bfloat16)
            contrib = lax.dot_general(
                ctx, woT_c, (((1,), (1,)), ((), ())),
                preferred_element_type=jnp.float32,
            )
            if h == 0:
                out_ref[...] = contrib
            else:
                out_ref[...] += contrib

            if h < N_DEV - 1:
                rdma.wait()

    out = pl.pallas_call(
        body,
        out_shape=jax.ShapeDtypeStruct((n_rows, D_MODEL), jnp.float32),
        in_specs=[
            pl.BlockSpec(memory_space=pltpu.VMEM),
            pl.BlockSpec(memory_space=pltpu.VMEM),
            pl.BlockSpec(memory_space=pltpu.VMEM),
            pl.BlockSpec(memory_space=pltpu.VMEM),
        ],
        out_specs=pl.BlockSpec(memory_space=pltpu.VMEM),
        scratch_shapes=[
            pltpu.VMEM((N_DEV, D_MODEL + hid_loc, hid_loc), jnp.bfloat16),
            pltpu.SemaphoreType.DMA((N_DEV,)),
            pltpu.SemaphoreType.DMA((N_DEV,)),
        ],
        compiler_params=pltpu.CompilerParams(collective_id=0),
    )(x2, payload, kt, vt)

    return out.reshape(B, SQ_LOC, D_MODEL)


# baseline (device time: 35629 ns/iter reference)
import jax
import jax.numpy as jnp
from jax import lax
from jax.experimental import pallas as pl
from jax.experimental.pallas import tpu as pltpu

N_DEV = 4
HQ_LOC = 4
DH = 64
SQ_LOC = 256
B = 2
D_MODEL = 512
D_HID = 1024


def kernel(x, Wq, K_ext, V_ext, Wo):
    x2 = x.reshape(B * SQ_LOC, D_MODEL).astype(jnp.bfloat16)
    payload = jnp.concatenate(
        [Wq.astype(jnp.bfloat16), Wo.T.astype(jnp.bfloat16)], axis=0
    )
    kt = jnp.transpose(K_ext, (2, 0, 1, 3)).reshape(16, B * SQ_LOC, DH)
    vt = jnp.transpose(V_ext, (2, 0, 1, 3)).reshape(16, B * SQ_LOC, DH)
    kt = kt.astype(jnp.bfloat16)
    vt = vt.astype(jnp.bfloat16)

    n_rows = B * SQ_LOC
    hid_loc = HQ_LOC * DH

    def body(x_ref, pay_ref, k_ref, v_ref, out_ref,
             comm_ref, send_sems, recv_sems):
        my = lax.axis_index("i")
        left = lax.rem(my + (N_DEV - 1), N_DEV)
        right = lax.rem(my + 1, N_DEV)

        barrier = pltpu.get_barrier_semaphore()
        for nbr in (left, right):
            pl.semaphore_signal(
                barrier, inc=1,
                device_id=(nbr,), device_id_type=pl.DeviceIdType.MESH,
            )
        pl.semaphore_wait(barrier, 2)

        comm_ref[0] = pay_ref[...]

        row = lax.broadcasted_iota(jnp.int32, (n_rows, n_rows), 0)
        col = lax.broadcasted_iota(jnp.int32, (n_rows, n_rows), 1)
        mask = ((row // SQ_LOC) == (col // SQ_LOC)) & (
            ((row // DH) % HQ_LOC) == ((col // DH) % HQ_LOC)
        )

        for h in range(N_DEV):
            if h < N_DEV - 1:
                rdma = pltpu.make_async_remote_copy(
                    src_ref=comm_ref.at[h],
                    dst_ref=comm_ref.at[h + 1],
                    send_sem=send_sems.at[h],
                    recv_sem=recv_sems.at[h + 1],
                    device_id=(right,),
                    device_id_type=pl.DeviceIdType.MESH,
                )
                rdma.start()

            origin = lax.rem(my - h + N_DEV, N_DEV)
            wq_c = comm_ref[h, :D_MODEL, :]
            woT_c = comm_ref[h, D_MODEL:, :]

            q_full = jnp.dot(
                x_ref[...], wq_c, preferred_element_type=jnp.float32
            )

            ctx_parts = []
            for t in range(HQ_LOC):
                head = origin * HQ_LOC + t
                q = q_full[:, t * DH:(t + 1) * DH].astype(jnp.bfloat16)
                k = k_ref[head]
                s = lax.dot_general(
                    q, k, (((1,), (1,)), ((), ())),
                    preferred_element_type=jnp.float32,
                ) * 0.125
                s = jnp.where(mask, s, -1e9)
                m = jnp.max(s, axis=1, keepdims=True)
                w = jnp.exp(s - m)
                w = w / jnp.sum(w, axis=1, keepdims=True)
                ctx_parts.append(
                    jnp.dot(
                        w.astype(jnp.bfloat16), v_ref[head],
                        preferred_element_type=jnp.float32,
                    )
                )
            ctx = jnp.concatenate(ctx_parts, axis=1).astype(jnp.bfloat16)
            contrib = lax.dot_general(
                ctx, woT_c, (((1,), (1,)), ((), ())),
                preferred_element_type=jnp.float32,
            )
            if h == 0:
                out_ref[...] = contrib
            else:
                out_ref[...] += contrib

            if h < N_DEV - 1:
                rdma.wait()

    out = pl.pallas_call(
        body,
        out_shape=jax.ShapeDtypeStruct((n_rows, D_MODEL), jnp.float32),
        in_specs=[
            pl.BlockSpec(memory_space=pltpu.VMEM),
            pl.BlockSpec(memory_space=pltpu.VMEM),
            pl.BlockSpec(memory_space=pltpu.VMEM),
            pl.BlockSpec(memory_space=pltpu.VMEM),
        ],
        out_specs=pl.BlockSpec(memory_space=pltpu.VMEM),
        scratch_shapes=[
            pltpu.VMEM((N_DEV, 2 * D_MODEL, hid_loc), jnp.bfloat16),
            pltpu.SemaphoreType.DMA((N_DEV,)),
            pltpu.SemaphoreType.DMA((N_DEV,)),
        ],
        compiler_params=pltpu.CompilerParams(collective_id=0),
    )(x2, payload, kt, vt)

    return out.reshape(B, SQ_LOC, D_MODEL)


# device time: 32037 ns/iter; 1.1121x vs baseline; 1.1121x over previous
import jax
import jax.numpy as jnp
from jax import lax
from jax.experimental import pallas as pl
from jax.experimental.pallas import tpu as pltpu

N_DEV = 4
HQ_LOC = 4
HQ = 16
DH = 64
SQ_LOC = 256
QB = 4
B = 2
D_MODEL = 512
ROWS = B * SQ_LOC
RB = B * 64
HID_LOC = HQ_LOC * DH
PAY_ROWS = 2 * D_MODEL
HALF = D_MODEL


def kernel(x, Wq, K_ext, V_ext, Wo):
    x2p = (
        x.reshape(B, QB, 64, D_MODEL)
        .transpose(1, 0, 2, 3)
        .reshape(ROWS, D_MODEL)
        .astype(jnp.bfloat16)
    )
    payload = jnp.concatenate(
        [(Wq * 0.125).astype(jnp.bfloat16), Wo.T.astype(jnp.bfloat16)],
        axis=0,
    )
    k_al = (
        K_ext.reshape(B, QB, 64, HQ, DH)
        .transpose(3, 1, 0, 2, 4)
        .reshape(HQ, QB, RB, DH)
        .astype(jnp.bfloat16)
    )
    v_al = (
        V_ext.reshape(B, QB, 64, HQ, DH)
        .transpose(3, 1, 0, 2, 4)
        .reshape(HQ, QB, RB, DH)
        .astype(jnp.bfloat16)
    )

    def body(x_ref, pay_ref, k_ref, v_ref, out_ref,
             comm_ref, ctx_ref, send_sems, recv_sems):
        my = lax.axis_index("i")
        left = lax.rem(my + (N_DEV - 1), N_DEV)
        right = lax.rem(my + 1, N_DEV)

        barrier = pltpu.get_barrier_semaphore()
        for nbr in (left, right):
            pl.semaphore_signal(
                barrier, inc=1,
                device_id=(nbr,), device_id_type=pl.DeviceIdType.MESH,
            )
        pl.semaphore_wait(barrier, 2)

        comm_ref[0] = pay_ref[...]

        row = lax.broadcasted_iota(jnp.int32, (RB, RB), 0)
        col = lax.broadcasted_iota(jnp.int32, (RB, RB), 1)
        maskf = ((row // 64) == (col // 64)).astype(jnp.float32)

        send_c_right = pltpu.make_async_remote_copy(
            src_ref=comm_ref.at[0], dst_ref=comm_ref.at[1],
            send_sem=send_sems.at[0], recv_sem=recv_sems.at[0],
            device_id=(right,), device_id_type=pl.DeviceIdType.MESH,
        )
        send_c_left = pltpu.make_async_remote_copy(
            src_ref=comm_ref.at[0], dst_ref=comm_ref.at[2],
            send_sem=send_sems.at[1], recv_sem=recv_sems.at[1],
            device_id=(left,), device_id_type=pl.DeviceIdType.MESH,
        )
        fwd_right = pltpu.make_async_remote_copy(
            src_ref=comm_ref.at[1, pl.ds(0, HALF)],
            dst_ref=comm_ref.at[3, pl.ds(0, HALF)],
            send_sem=send_sems.at[2], recv_sem=recv_sems.at[2],
            device_id=(right,), device_id_type=pl.DeviceIdType.MESH,
        )
        fwd_left = pltpu.make_async_remote_copy(
            src_ref=comm_ref.at[2, pl.ds(HALF, HALF)],
            dst_ref=comm_ref.at[3, pl.ds(HALF, HALF)],
            send_sem=send_sems.at[3], recv_sem=recv_sems.at[3],
            device_id=(left,), device_id_type=pl.DeviceIdType.MESH,
        )

        def compute(slot, origin, first):
            wq_c = comm_ref[slot, :D_MODEL, :]
            woT_c = comm_ref[slot, D_MODEL:, :]
            q_bf = jnp.dot(
                x_ref[...], wq_c, preferred_element_type=jnp.float32
            ).astype(jnp.bfloat16)
            for p in range(QB):
                for t in range(HQ_LOC):
                    head = origin * HQ_LOC + t
                    q = q_bf[p * RB:(p + 1) * RB, t * DH:(t + 1) * DH]
                    k = k_ref[head, p]
                    s = lax.dot_general(
                        q, k, (((1,), (1,)), ((), ())),
                        preferred_element_type=jnp.float32,
                    )
                    e = jnp.exp(s) * maskf
                    denom = jnp.sum(e, axis=1, keepdims=True)
                    ctx = jnp.dot(
                        e.astype(jnp.bfloat16), v_ref[head, p],
                        preferred_element_type=jnp.float32,
                    ) * (1.0 / denom)
                    ctx_ref[p * RB:(p + 1) * RB, t * DH:(t + 1) * DH] = (
                        ctx.astype(jnp.bfloat16)
                    )
            contrib = lax.dot_general(
                ctx_ref[...], woT_c, (((1,), (1,)), ((), ())),
                preferred_element_type=jnp.float32,
            )
            if first:
                out_ref[...] = contrib
            else:
                out_ref[...] += contrib

        send_c_right.start()
        send_c_left.start()
        compute(0, my, first=True)

        send_c_right.wait_recv()
        fwd_right.start()
        compute(1, lax.rem(my + (N_DEV - 1), N_DEV), first=False)
        fwd_right.wait_send()

        send_c_left.wait_recv()
        fwd_left.start()
        compute(2, lax.rem(my + 1, N_DEV), first=False)

        fwd_right.wait_recv()
        fwd_left.wait_recv()
        compute(3, lax.rem(my + 2, N_DEV), first=False)

        send_c_right.wait_send()
        send_c_left.wait_send()
        fwd_left.wait_send()

    out = pl.pallas_call(
        body,
        out_shape=jax.ShapeDtypeStruct((ROWS, D_MODEL), jnp.float32),
        in_specs=[
            pl.BlockSpec(memory_space=pltpu.VMEM),
            pl.BlockSpec(memory_space=pltpu.VMEM),
            pl.BlockSpec(memory_space=pltpu.VMEM),
            pl.BlockSpec(memory_space=pltpu.VMEM),
        ],
        out_specs=pl.BlockSpec(memory_space=pltpu.VMEM),
        scratch_shapes=[
            pltpu.VMEM((N_DEV, PAY_ROWS, HID_LOC), jnp.bfloat16),
            pltpu.VMEM((ROWS, HID_LOC), jnp.bfloat16),
            pltpu.SemaphoreType.DMA((4,)),
            pltpu.SemaphoreType.DMA((4,)),
        ],
        compiler_params=pltpu.CompilerParams(collective_id=0),
    )(x2p, payload, k_al, v_al)

    return (
        out.reshape(QB, B, 64, D_MODEL)
        .transpose(1, 0, 2, 3)
        .reshape(B, SQ_LOC, D_MODEL)
    )
